# baseline (speedup 1.0000x reference)
import functools

import jax
import jax.numpy as jnp
from jax import lax
from jax.experimental import pallas as pl
from jax.experimental.pallas import tpu as pltpu
from jax.experimental.pallas import tpu_sc as plsc

N_ROWS = 1000000
FEATS = 16
BATCH_C = 16384

_info = plsc.get_sparse_core_info()
NC = _info.num_cores
NS = _info.num_subcores
LANES = _info.num_lanes
NW = NC * NS
B_PER_W = BATCH_C // NW
CHUNK = 128
N_BIAS_CHUNKS = B_PER_W // CHUNK
N_EMB_CHUNKS = B_PER_W * FEATS // CHUNK
GROUPS = B_PER_W // LANES


def _mf_kernel(user_hbm, item_hbm, ub_hbm, ib_hbm, ue_hbm, ie_hbm, out_hbm,
               uidx_v, iidx_v, gu_v, gi_v, du_v, di_v, ub_v, ib_v, out_v, sem):
    wid = lax.axis_index("s") * NC + lax.axis_index("c")
    base = wid * B_PER_W

    pltpu.sync_copy(user_hbm.at[pl.ds(base, B_PER_W)], uidx_v)
    pltpu.sync_copy(item_hbm.at[pl.ds(base, B_PER_W)], iidx_v)

    def build_body(g, _):
        e0 = g * LANES
        u16 = uidx_v[pl.ds(e0, LANES)]
        i16 = iidx_v[pl.ds(e0, LANES)]
        for f in range(FEATS):
            gu_v[pl.ds(f * B_PER_W + e0, LANES)] = u16 + f * N_ROWS
            gi_v[pl.ds(f * B_PER_W + e0, LANES)] = i16 + f * N_ROWS
        return 0

    lax.fori_loop(0, GROUPS, build_body, 0)

    copies = []
    for c in range(N_BIAS_CHUNKS):
        s = pl.ds(c * CHUNK, CHUNK)
        copies.append(pltpu.make_async_copy(ub_hbm.at[uidx_v.at[s]], ub_v.at[s], sem))
        copies.append(pltpu.make_async_copy(ib_hbm.at[iidx_v.at[s]], ib_v.at[s], sem))
    for c in range(N_EMB_CHUNKS):
        s = pl.ds(c * CHUNK, CHUNK)
        copies.append(pltpu.make_async_copy(ue_hbm.at[gu_v.at[s]], du_v.at[s], sem))
        copies.append(pltpu.make_async_copy(ie_hbm.at[gi_v.at[s]], di_v.at[s], sem))
    for cp in copies:
        cp.start()
    for cp in copies:
        cp.wait()

    def group_body(g, _):
        e0 = g * LANES
        acc = ub_v[pl.ds(e0, LANES)] + ib_v[pl.ds(e0, LANES)]
        for f in range(FEATS):
            acc = acc + du_v[pl.ds(f * B_PER_W + e0, LANES)] * di_v[pl.ds(f * B_PER_W + e0, LANES)]
        out_v[pl.ds(e0, LANES)] = acc
        return 0

    lax.fori_loop(0, GROUPS, group_body, 0)

    pltpu.sync_copy(out_v, out_hbm.at[pl.ds(base, B_PER_W)])


@jax.jit
def _mf(user, item, u_bias_flat, i_bias_flat, ue_flat, ie_flat):
    mesh = plsc.VectorSubcoreMesh(core_axis_name="c", subcore_axis_name="s")
    return pl.kernel(
        _mf_kernel,
        out_type=jax.ShapeDtypeStruct((BATCH_C,), jnp.float32),
        mesh=mesh,
        compiler_params=pltpu.CompilerParams(needs_layout_passes=False),
        scratch_types=[
            pltpu.VMEM((B_PER_W,), jnp.int32),
            pltpu.VMEM((B_PER_W,), jnp.int32),
            pltpu.VMEM((B_PER_W * FEATS,), jnp.int32),
            pltpu.VMEM((B_PER_W * FEATS,), jnp.int32),
            pltpu.VMEM((B_PER_W * FEATS,), jnp.float32),
            pltpu.VMEM((B_PER_W * FEATS,), jnp.float32),
            pltpu.VMEM((B_PER_W,), jnp.float32),
            pltpu.VMEM((B_PER_W,), jnp.float32),
            pltpu.VMEM((B_PER_W,), jnp.float32),
            pltpu.SemaphoreType.DMA,
        ],
    )(user, item, u_bias_flat, i_bias_flat, ue_flat, ie_flat)


def kernel(user, item, u_bias, i_bias, u_embed, i_embed):
    return _mf(
        user.astype(jnp.int32),
        item.astype(jnp.int32),
        u_bias.reshape(-1),
        i_bias.reshape(-1),
        u_embed.T.reshape(-1),
        i_embed.T.reshape(-1),
    )


# speedup vs baseline: 3.2199x; 3.2199x over previous
import functools

import jax
import jax.numpy as jnp
from jax import lax
from jax.experimental import pallas as pl
from jax.experimental.pallas import tpu as pltpu
from jax.experimental.pallas import tpu_sc as plsc

N_ROWS = 1000000
FEATS = 16
BATCH_C = 16384

_info = plsc.get_sparse_core_info()
NC = _info.num_cores
NS = _info.num_subcores
LANES = _info.num_lanes
NW = NC * NS
B_PER_W = BATCH_C // NW
CHUNK = 128
N_BIAS_CHUNKS = B_PER_W // CHUNK
N_EMB_CHUNKS = B_PER_W * FEATS // CHUNK
GROUPS = B_PER_W // LANES


def _mf_kernel(user_hbm, item_hbm, ub_hbm, ib_hbm, ue_hbm, ie_hbm, out_hbm,
               uidx_v, iidx_v, gu_v, gi_v, du_v, di_v, ub_v, ib_v, out_v, sem):
    wid = lax.axis_index("s") * NC + lax.axis_index("c")
    base = wid * B_PER_W

    pltpu.sync_copy(user_hbm.at[pl.ds(base, B_PER_W)], uidx_v)
    pltpu.sync_copy(item_hbm.at[pl.ds(base, B_PER_W)], iidx_v)

    def build_body(g, _):
        e0 = g * LANES
        u16 = uidx_v[pl.ds(e0, LANES)] * FEATS
        i16 = iidx_v[pl.ds(e0, LANES)] * FEATS
        for f in range(FEATS):
            gu_v[pl.ds(f * B_PER_W + e0, LANES)] = u16 + f
            gi_v[pl.ds(f * B_PER_W + e0, LANES)] = i16 + f
        return 0

    lax.fori_loop(0, GROUPS, build_body, 0)

    copies = []
    for c in range(N_BIAS_CHUNKS):
        s = pl.ds(c * CHUNK, CHUNK)
        copies.append(pltpu.make_async_copy(ub_hbm.at[uidx_v.at[s]], ub_v.at[s], sem))
        copies.append(pltpu.make_async_copy(ib_hbm.at[iidx_v.at[s]], ib_v.at[s], sem))
    for c in range(N_EMB_CHUNKS):
        s = pl.ds(c * CHUNK, CHUNK)
        copies.append(pltpu.make_async_copy(ue_hbm.at[gu_v.at[s]], du_v.at[s], sem))
        copies.append(pltpu.make_async_copy(ie_hbm.at[gi_v.at[s]], di_v.at[s], sem))
    for cp in copies:
        cp.start()
    for cp in copies:
        cp.wait()

    def group_body(g, _):
        e0 = g * LANES
        acc = ub_v[pl.ds(e0, LANES)] + ib_v[pl.ds(e0, LANES)]
        for f in range(FEATS):
            acc = acc + du_v[pl.ds(f * B_PER_W + e0, LANES)] * di_v[pl.ds(f * B_PER_W + e0, LANES)]
        out_v[pl.ds(e0, LANES)] = acc
        return 0

    lax.fori_loop(0, GROUPS, group_body, 0)

    pltpu.sync_copy(out_v, out_hbm.at[pl.ds(base, B_PER_W)])


@jax.jit
def _mf(user, item, u_bias_flat, i_bias_flat, ue_flat, ie_flat):
    mesh = plsc.VectorSubcoreMesh(core_axis_name="c", subcore_axis_name="s")
    return pl.kernel(
        _mf_kernel,
        out_type=jax.ShapeDtypeStruct((BATCH_C,), jnp.float32),
        mesh=mesh,
        compiler_params=pltpu.CompilerParams(needs_layout_passes=False),
        scratch_types=[
            pltpu.VMEM((B_PER_W,), jnp.int32),
            pltpu.VMEM((B_PER_W,), jnp.int32),
            pltpu.VMEM((B_PER_W * FEATS,), jnp.int32),
            pltpu.VMEM((B_PER_W * FEATS,), jnp.int32),
            pltpu.VMEM((B_PER_W * FEATS,), jnp.float32),
            pltpu.VMEM((B_PER_W * FEATS,), jnp.float32),
            pltpu.VMEM((B_PER_W,), jnp.float32),
            pltpu.VMEM((B_PER_W,), jnp.float32),
            pltpu.VMEM((B_PER_W,), jnp.float32),
            pltpu.SemaphoreType.DMA,
        ],
    )(user, item, u_bias_flat, i_bias_flat, ue_flat, ie_flat)


def kernel(user, item, u_bias, i_bias, u_embed, i_embed):
    return _mf(
        user.astype(jnp.int32),
        item.astype(jnp.int32),
        u_bias.reshape(-1),
        i_bias.reshape(-1),
        u_embed.reshape(-1),
        i_embed.reshape(-1),
    )


# speedup vs baseline: 12.2024x; 3.7897x over previous
import functools

import jax
import jax.numpy as jnp
from jax import lax
from jax.experimental import pallas as pl
from jax.experimental.pallas import tpu as pltpu
from jax.experimental.pallas import tpu_sc as plsc

N_ROWS = 1000000
FEATS = 16
BATCH_C = 16384
SLAB = 128

_info = plsc.get_sparse_core_info()
NC = _info.num_cores
NS = _info.num_subcores
LANES = _info.num_lanes
NW = NC * NS
B_PER_W = BATCH_C // NW
CHUNK = 128
N_BIAS_CHUNKS = B_PER_W // CHUNK
GROUPS = B_PER_W // LANES


def _mf_kernel(user_hbm, item_hbm, ub_hbm, ib_hbm, ue_hbm, ie_hbm, out_hbm,
               uidx_v, iidx_v, uslab_v, islab_v, ub_v, ib_v, out_v, sem, bsem):
    wid = lax.axis_index("s") * NC + lax.axis_index("c")
    base = wid * B_PER_W

    pltpu.sync_copy(user_hbm.at[pl.ds(base, B_PER_W)], uidx_v)
    pltpu.sync_copy(item_hbm.at[pl.ds(base, B_PER_W)], iidx_v)

    bias_copies = []
    for c in range(N_BIAS_CHUNKS):
        s = pl.ds(c * CHUNK, CHUNK)
        bias_copies.append(pltpu.make_async_copy(ub_hbm.at[uidx_v.at[s]], ub_v.at[s], bsem))
        bias_copies.append(pltpu.make_async_copy(ib_hbm.at[iidx_v.at[s]], ib_v.at[s], bsem))
    for cp in bias_copies:
        cp.start()

    lane_iota = lax.broadcasted_iota(jnp.int32, (LANES,), 0)

    def group_body(g, _):
        e0 = g * LANES
        u16 = uidx_v[pl.ds(e0, LANES)]
        i16 = iidx_v[pl.ds(e0, LANES)]
        chunk_copies = []
        for l in range(LANES):
            cu = pl.multiple_of((u16[l] // SLAB) * SLAB, SLAB)
            ci = pl.multiple_of((i16[l] // SLAB) * SLAB, SLAB)
            chunk_copies.append(pltpu.make_async_copy(
                ue_hbm.at[:, pl.ds(cu, SLAB)], uslab_v.at[l], sem))
            chunk_copies.append(pltpu.make_async_copy(
                ie_hbm.at[:, pl.ds(ci, SLAB)], islab_v.at[l], sem))
        for cp in chunk_copies:
            cp.start()
        for cp in chunk_copies:
            cp.wait()

        cu16 = lax.rem(u16, SLAB)
        ci16 = lax.rem(i16, SLAB)
        acc = jnp.zeros((LANES,), jnp.float32)
        for f in range(FEATS):
            fvec = jnp.full((LANES,), f, jnp.int32)
            vu = plsc.load_gather(uslab_v, [lane_iota, fvec, cu16])
            vi = plsc.load_gather(islab_v, [lane_iota, fvec, ci16])
            acc = acc + vu * vi
        out_v[pl.ds(e0, LANES)] = acc
        return 0

    lax.fori_loop(0, GROUPS, group_body, 0)

    for cp in bias_copies:
        cp.wait()

    def bias_body(g, _):
        e0 = g * LANES
        out_v[pl.ds(e0, LANES)] = (
            out_v[pl.ds(e0, LANES)] + ub_v[pl.ds(e0, LANES)] + ib_v[pl.ds(e0, LANES)]
        )
        return 0

    lax.fori_loop(0, GROUPS, bias_body, 0)

    pltpu.sync_copy(out_v, out_hbm.at[pl.ds(base, B_PER_W)])


@jax.jit
def _mf(user, item, u_bias_flat, i_bias_flat, ue_t, ie_t):
    mesh = plsc.VectorSubcoreMesh(core_axis_name="c", subcore_axis_name="s")
    return pl.kernel(
        _mf_kernel,
        out_type=jax.ShapeDtypeStruct((BATCH_C,), jnp.float32),
        mesh=mesh,
        compiler_params=pltpu.CompilerParams(needs_layout_passes=False, use_tc_tiling_on_sc=True),
        scratch_types=[
            pltpu.VMEM((B_PER_W,), jnp.int32),
            pltpu.VMEM((B_PER_W,), jnp.int32),
            pltpu.VMEM((LANES, FEATS, SLAB), jnp.float32),
            pltpu.VMEM((LANES, FEATS, SLAB), jnp.float32),
            pltpu.VMEM((B_PER_W,), jnp.float32),
            pltpu.VMEM((B_PER_W,), jnp.float32),
            pltpu.VMEM((B_PER_W,), jnp.float32),
            pltpu.SemaphoreType.DMA,
            pltpu.SemaphoreType.DMA,
        ],
    )(user, item, u_bias_flat, i_bias_flat, ue_t, ie_t)


def kernel(user, item, u_bias, i_bias, u_embed, i_embed):
    return _mf(
        user.astype(jnp.int32),
        item.astype(jnp.int32),
        u_bias.reshape(-1),
        i_bias.reshape(-1),
        u_embed.T,
        i_embed.T,
    )
